# baseline (speedup 1.0000x reference)
import functools

import jax
import jax.numpy as jnp
from jax import lax
from jax.experimental import pallas as pl
from jax.experimental.pallas import tpu as pltpu
from jax.experimental.pallas import tpu_sc as plsc

D = 1024
GROUP = 64
L = 16
CH = 16
NSLOT = 3
UNROLL = 8


def _sc_body(x_hbm, sel_hbm, pe_hbm, out_hbm,
             pe_v, sel_v, perm_v, pev_v,
             *scratch, rows_per_worker):
    c = rows_per_worker
    ng = c // L
    bufs = list(scratch[0:NSLOT])
    ssts = list(scratch[NSLOT:2 * NSLOT])
    tsts = list(scratch[2 * NSLOT:3 * NSLOT])
    gsems = list(scratch[3 * NSLOT:4 * NSLOT])
    ssems = list(scratch[4 * NSLOT:5 * NSLOT])

    nc = 2
    wid = lax.axis_index("s") * nc + lax.axis_index("c")
    base = wid * c
    pltpu.sync_copy(pe_hbm, pe_v)
    pltpu.sync_copy(sel_hbm.at[pl.ds(base, c)], sel_v)
    iota = lax.iota(jnp.int32, L)

    def comp_body(j, carry):
        om, ou = carry
        vsel = sel_v[pl.ds(j * L, L)]
        vm = vsel >= 0
        vm32 = vm.astype(jnp.int32)
        inc = plsc.cumsum(vm32)
        incu = plsc.cumsum(1 - vm32)
        pos_m = inc + (om - 1)
        pos_u = (ou + 1) - incu
        rows = iota + (base + j * L)
        plsc.store_scatter(perm_v, [pos_m], rows, mask=vm)
        plsc.store_scatter(pev_v, [pos_m], vsel, mask=vm)
        plsc.store_scatter(perm_v, [pos_u], rows, mask=jnp.logical_not(vm))
        cm = jnp.sum(vm32)
        return om + cm, ou - (L - cm)

    nm, _ = lax.fori_loop(0, ng, comp_body,
                          (jnp.int32(0), jnp.int32(c - 1)))
    nu = c - nm

    def run_pipeline(nchunks, stage_fn, gather_src):
        def gd(s):
            return pltpu.make_async_copy(
                gather_src.at[ssts[s]], bufs[s], gsems[s])

        def sd(s):
            return pltpu.make_async_copy(bufs[s], out_hbm.at[tsts[s]], ssems[s])

        nsup = (nchunks + NSLOT - 1) // NSLOT + 1

        def sup_body(h, _):
            for b in range(NSLOT):
                t = h * NSLOT + b
                @pl.when(jnp.logical_and(t >= NSLOT, t - NSLOT < nchunks))
                def _():
                    sd(b).wait()
                if gather_src is not None:
                    @pl.when(t < nchunks)
                    def _():
                        stage_fn(t, b)
                        gd(b).start()
                else:
                    @pl.when(t < nchunks)
                    def _():
                        stage_fn(t, b)
                        sd(b).start()
            if gather_src is not None:
                for b in range(NSLOT):
                    t = h * NSLOT + b
                    @pl.when(t < nchunks)
                    def _():
                        gd(b).wait()
                        sd(b).start()
            return 0

        lax.fori_loop(0, nsup, sup_body, 0)

    fill_m = jnp.full((L,), jnp.maximum(nm - 1, 0), jnp.int32)
    fillp = plsc.load_gather(pev_v, [fill_m])
    fillt_m = plsc.load_gather(perm_v, [fill_m])
    nch_m = (nm + CH - 1) // CH
    wmax_m = jnp.maximum(nm - CH, 0)

    def stage_m(t, s):
        w = jnp.minimum(t * CH, wmax_m)
        q = iota < (nm - w)
        pv = jnp.where(q, pev_v[pl.ds(w, L)], fillp)
        tsts[s][...] = jnp.where(q, perm_v[pl.ds(w, L)], fillt_m)
        buf = bufs[s]

        @plsc.parallel_loop(0, D, 1, unroll=UNROLL)
        def _(k):
            off = jnp.full((L,), k, jnp.int32)
            vals = plsc.load_gather(pe_v, [pv, off])
            plsc.store_scatter(buf, [iota, off], vals)

    run_pipeline(nch_m, stage_m, None)

    fill_u = jnp.full((L,), c - 1, jnp.int32)
    fillt_u = plsc.load_gather(perm_v, [fill_u])
    nch_u = (nu + CH - 1) // CH

    def stage_u(t, s):
        w = jnp.minimum(nm + t * CH, c - CH)
        q = iota >= (nm - w)
        v = jnp.where(q, perm_v[pl.ds(w, L)], fillt_u)
        ssts[s][...] = v
        tsts[s][...] = v

    run_pipeline(nch_u, stage_u, x_hbm)


def kernel(x, local_indices, group_mask, pe):
    b, s, d = x.shape
    n = b * s
    nw = 32
    c = n // nw
    x2 = x.reshape(n, d)
    sel = jnp.where(group_mask, local_indices, -1).reshape(n)

    mesh = plsc.VectorSubcoreMesh(core_axis_name="c", subcore_axis_name="s")
    sc_kernel = functools.partial(
        pl.kernel,
        out_type=jax.ShapeDtypeStruct((n, d), jnp.float32),
        mesh=mesh,
        compiler_params=pltpu.CompilerParams(needs_layout_passes=False),
        scratch_types=(
            [pltpu.VMEM((GROUP, d), jnp.float32)]
            + [pltpu.VMEM((c,), jnp.int32)] * 3
            + [pltpu.VMEM((CH, d), jnp.float32)] * NSLOT
            + [pltpu.VMEM((CH,), jnp.int32)] * (2 * NSLOT)
            + [pltpu.SemaphoreType.DMA] * (2 * NSLOT)
        ),
    )(functools.partial(_sc_body, rows_per_worker=c))
    out = sc_kernel(x2, sel, pe)
    return out.reshape(b, s, d)


# speedup vs baseline: 3.6021x; 3.6021x over previous
import functools

import jax
import jax.numpy as jnp
from jax import lax
from jax.experimental import pallas as pl
from jax.experimental.pallas import tpu as pltpu
from jax.experimental.pallas import tpu_sc as plsc

D = 1024
GROUP = 64
L = 16
CH = 16
NSLOT = 3
UNROLL = 8


def _sc_body(x_hbm, sel_hbm, pe_hbm, out_hbm,
             pe_v, sel_v, perm_v, pev_v,
             *scratch, rows_per_worker):
    c = rows_per_worker
    ng = c // L
    bufs = list(scratch[0:NSLOT])
    ssts = list(scratch[NSLOT:2 * NSLOT])
    tsts = list(scratch[2 * NSLOT:3 * NSLOT])
    gsems = list(scratch[3 * NSLOT:4 * NSLOT])
    ssems = list(scratch[4 * NSLOT:5 * NSLOT])

    nc = 2
    wid = lax.axis_index("s") * nc + lax.axis_index("c")
    base = wid * c
    pltpu.sync_copy(pe_hbm, pe_v)
    pltpu.sync_copy(sel_hbm.at[pl.ds(base, c)], sel_v)
    iota = lax.iota(jnp.int32, L)

    def comp_body(j, carry):
        om, ou = carry
        vsel = sel_v[pl.ds(j * L, L)]
        vm = vsel >= 0
        vm32 = vm.astype(jnp.int32)
        inc = plsc.cumsum(vm32)
        incu = plsc.cumsum(1 - vm32)
        pos_m = inc + (om - 1)
        pos_u = (ou + 1) - incu
        rows = iota + (base + j * L)
        plsc.store_scatter(perm_v, [pos_m], rows, mask=vm)
        plsc.store_scatter(pev_v, [pos_m], vsel, mask=vm)
        plsc.store_scatter(perm_v, [pos_u], rows, mask=jnp.logical_not(vm))
        cm = jnp.sum(vm32)
        return om + cm, ou - (L - cm)

    nm, _ = lax.fori_loop(0, ng, comp_body,
                          (jnp.int32(0), jnp.int32(c - 1)))
    nu = c - nm

    def run_pipeline(nchunks, stage_fn, gather_src):
        def gd(s):
            return pltpu.make_async_copy(
                gather_src.at[ssts[s]], bufs[s], gsems[s])

        def sd(s):
            return pltpu.make_async_copy(bufs[s], out_hbm.at[tsts[s]], ssems[s])

        nsup = (nchunks + NSLOT - 1) // NSLOT + 1

        def sup_body(h, _):
            for b in range(NSLOT):
                t = h * NSLOT + b
                @pl.when(jnp.logical_and(t >= NSLOT, t - NSLOT < nchunks))
                def _():
                    sd(b).wait()
                if gather_src is not None:
                    @pl.when(t < nchunks)
                    def _():
                        stage_fn(t, b)
                        gd(b).start()
                else:
                    @pl.when(t < nchunks)
                    def _():
                        stage_fn(t, b)
                        sd(b).start()
            if gather_src is not None:
                for b in range(NSLOT):
                    t = h * NSLOT + b
                    @pl.when(t < nchunks)
                    def _():
                        gd(b).wait()
                        sd(b).start()
            return 0

        lax.fori_loop(0, nsup, sup_body, 0)

    fill_m = jnp.full((L,), jnp.maximum(nm - 1, 0), jnp.int32)
    fillp = plsc.load_gather(pev_v, [fill_m])
    fillt_m = plsc.load_gather(perm_v, [fill_m])
    nch_m = (nm + CH - 1) // CH
    wmax_m = jnp.maximum(nm - CH, 0)

    def stage_m(t, s):
        w = jnp.minimum(t * CH, wmax_m)
        q = iota < (nm - w)
        pv = jnp.where(q, pev_v[pl.ds(w, L)], fillp)
        tsts[s][...] = jnp.where(q, perm_v[pl.ds(w, L)], fillt_m)
        buf = bufs[s]

        diag = iota * 65

        @plsc.parallel_loop(0, D, 1, unroll=UNROLL)
        def _(k):
            off = (diag + k) & (D - 1)
            vals = plsc.load_gather(pe_v, [pv, off])
            plsc.store_scatter(buf, [iota, off], vals)

    run_pipeline(nch_m, stage_m, None)

    fill_u = jnp.full((L,), c - 1, jnp.int32)
    fillt_u = plsc.load_gather(perm_v, [fill_u])
    nch_u = (nu + CH - 1) // CH

    def stage_u(t, s):
        w = jnp.minimum(nm + t * CH, c - CH)
        q = iota >= (nm - w)
        v = jnp.where(q, perm_v[pl.ds(w, L)], fillt_u)
        ssts[s][...] = v
        tsts[s][...] = v

    run_pipeline(nch_u, stage_u, x_hbm)


def kernel(x, local_indices, group_mask, pe):
    b, s, d = x.shape
    n = b * s
    nw = 32
    c = n // nw
    x2 = x.reshape(n, d)
    sel = jnp.where(group_mask, local_indices, -1).reshape(n)

    mesh = plsc.VectorSubcoreMesh(core_axis_name="c", subcore_axis_name="s")
    sc_kernel = functools.partial(
        pl.kernel,
        out_type=jax.ShapeDtypeStruct((n, d), jnp.float32),
        mesh=mesh,
        compiler_params=pltpu.CompilerParams(needs_layout_passes=False),
        scratch_types=(
            [pltpu.VMEM((GROUP, d), jnp.float32)]
            + [pltpu.VMEM((c,), jnp.int32)] * 3
            + [pltpu.VMEM((CH, d), jnp.float32)] * NSLOT
            + [pltpu.VMEM((CH,), jnp.int32)] * (2 * NSLOT)
            + [pltpu.SemaphoreType.DMA] * (2 * NSLOT)
        ),
    )(functools.partial(_sc_body, rows_per_worker=c))
    out = sc_kernel(x2, sel, pe)
    return out.reshape(b, s, d)


# speedup vs baseline: 4.0135x; 1.1142x over previous
import functools

import jax
import jax.numpy as jnp
from jax import lax
from jax.experimental import pallas as pl
from jax.experimental.pallas import tpu as pltpu
from jax.experimental.pallas import tpu_sc as plsc

D = 1024
GROUP = 64
L = 16
CH = 16
NSLOT = 3
UNROLL = 8


def _sc_body(x_hbm, sel_hbm, pe_hbm, out_hbm,
             pe_v, sel_v, perm_v, pev_v,
             *scratch, rows_per_worker):
    c = rows_per_worker
    ng = c // L
    bufs = list(scratch[0:NSLOT])
    ssts = list(scratch[NSLOT:2 * NSLOT])
    tsts = list(scratch[2 * NSLOT:3 * NSLOT])
    gsems = list(scratch[3 * NSLOT:4 * NSLOT])
    ssems = list(scratch[4 * NSLOT:5 * NSLOT])

    nc = 2
    wid = lax.axis_index("s") * nc + lax.axis_index("c")
    base = wid * c
    pltpu.sync_copy(pe_hbm, pe_v)
    pltpu.sync_copy(sel_hbm.at[pl.ds(base, c)], sel_v)
    iota = lax.iota(jnp.int32, L)

    def comp_body(j, carry):
        om, ou = carry
        vsel = sel_v[pl.ds(j * L, L)]
        vm = vsel >= 0
        vm32 = vm.astype(jnp.int32)
        inc = plsc.cumsum(vm32)
        incu = plsc.cumsum(1 - vm32)
        pos_m = inc + (om - 1)
        pos_u = (ou + 1) - incu
        rows = iota + (base + j * L)
        plsc.store_scatter(perm_v, [pos_m], rows, mask=vm)
        plsc.store_scatter(pev_v, [pos_m], vsel, mask=vm)
        plsc.store_scatter(perm_v, [pos_u], rows, mask=jnp.logical_not(vm))
        cm = jnp.sum(vm32)
        return om + cm, ou - (L - cm)

    nm, _ = lax.fori_loop(0, ng, comp_body,
                          (jnp.int32(0), jnp.int32(c - 1)))
    nu = c - nm

    fill_m = jnp.full((L,), jnp.maximum(nm - 1, 0), jnp.int32)
    fillp = plsc.load_gather(pev_v, [fill_m])
    fillt_m = plsc.load_gather(perm_v, [fill_m])
    nch_m = (nm + CH - 1) // CH
    wmax_m = jnp.maximum(nm - CH, 0)
    diag = iota * 65

    def stage_m(t, s):
        w = jnp.minimum(t * CH, wmax_m)
        q = iota < (nm - w)
        pv = jnp.where(q, pev_v[pl.ds(w, L)], fillp)
        tsts[s][...] = jnp.where(q, perm_v[pl.ds(w, L)], fillt_m)
        buf = bufs[s]

        @plsc.parallel_loop(0, D, 1, unroll=UNROLL)
        def _(k):
            off = (diag + k) & (D - 1)
            vals = plsc.load_gather(pe_v, [pv, off])
            plsc.store_scatter(buf, [iota, off], vals)

    fill_u = jnp.full((L,), c - 1, jnp.int32)
    fillt_u = plsc.load_gather(perm_v, [fill_u])
    nch_u = (nu + CH - 1) // CH

    def stage_u(t, s):
        w = jnp.minimum(nm + t * CH, c - CH)
        q = iota >= (nm - w)
        v = jnp.where(q, perm_v[pl.ds(w, L)], fillt_u)
        ssts[s][...] = v
        tsts[s][...] = v

    nch = nch_m + nch_u
    minc = jnp.minimum(nch_m, nch_u)
    min2 = 2 * minc

    def gd(s):
        return pltpu.make_async_copy(x_hbm.at[ssts[s]], bufs[s], gsems[s])

    def sd(s):
        return pltpu.make_async_copy(bufs[s], out_hbm.at[tsts[s]], ssems[s])

    nsup = (nch + NSLOT - 1) // NSLOT + 1

    def sup_body(h, _):
        for b in range(NSLOT):
            t = h * NSLOT + b
            is_m = jnp.where(t < min2, (t % 2) == 0, nch_m > nch_u)
            ordinal = jnp.where(t < min2, t // 2, t - min2 + minc)
            live = t < nch
            @pl.when(jnp.logical_and(t >= NSLOT, t - NSLOT < nch))
            def _():
                sd(b).wait()
            @pl.when(jnp.logical_and(live, is_m))
            def _():
                stage_m(ordinal, b)
                sd(b).start()
            @pl.when(jnp.logical_and(live, jnp.logical_not(is_m)))
            def _():
                stage_u(ordinal, b)
                gd(b).start()
        for b in range(NSLOT):
            t = h * NSLOT + b
            is_m = jnp.where(t < min2, (t % 2) == 0, nch_m > nch_u)
            @pl.when(jnp.logical_and(t < nch, jnp.logical_not(is_m)))
            def _():
                gd(b).wait()
                sd(b).start()
        return 0

    lax.fori_loop(0, nsup, sup_body, 0)


def kernel(x, local_indices, group_mask, pe):
    b, s, d = x.shape
    n = b * s
    nw = 32
    c = n // nw
    x2 = x.reshape(n, d)
    sel = jnp.where(group_mask, local_indices, -1).reshape(n)

    mesh = plsc.VectorSubcoreMesh(core_axis_name="c", subcore_axis_name="s")
    sc_kernel = functools.partial(
        pl.kernel,
        out_type=jax.ShapeDtypeStruct((n, d), jnp.float32),
        mesh=mesh,
        compiler_params=pltpu.CompilerParams(needs_layout_passes=False),
        scratch_types=(
            [pltpu.VMEM((GROUP, d), jnp.float32)]
            + [pltpu.VMEM((c,), jnp.int32)] * 3
            + [pltpu.VMEM((CH, d), jnp.float32)] * NSLOT
            + [pltpu.VMEM((CH,), jnp.int32)] * (2 * NSLOT)
            + [pltpu.SemaphoreType.DMA] * (2 * NSLOT)
        ),
    )(functools.partial(_sc_body, rows_per_worker=c))
    out = sc_kernel(x2, sel, pe)
    return out.reshape(b, s, d)
